# baseline (speedup 1.0000x reference)
import functools

import jax
import jax.numpy as jnp
from jax import lax
from jax.experimental import pallas as pl
from jax.experimental.pallas import tpu as pltpu
from jax.experimental.pallas import tpu_sc as plsc

V = 1000000
D = 64
B = 8192
NC, NS, L = 2, 16, 16
NW = NC * NS
BPW = B // NW
HALF = 1 << 19
RB = 4096
GRID = HALF // RB
NCB = (V + RB - 1) // RB - 1


def _transpose_pack(wt_T):
    def body(lo_ref, hi_ref, out_ref):
        eye = (jax.lax.broadcasted_iota(jnp.int32, (D, D), 0)
               == jax.lax.broadcasted_iota(jnp.int32, (D, D), 1)
               ).astype(jnp.float32)
        dn = (((0,), (0,)), ((), ()))
        lo_t = jax.lax.dot_general(lo_ref[...], eye, dn,
                                   preferred_element_type=jnp.float32)
        hi_t = jax.lax.dot_general(hi_ref[...], eye, dn,
                                   preferred_element_type=jnp.float32)
        out_ref[...] = jnp.concatenate([lo_t, hi_t], axis=1)

    return pl.pallas_call(
        body,
        grid=(GRID,),
        in_specs=[
            pl.BlockSpec((D, RB), lambda b: (0, b)),
            pl.BlockSpec((D, RB), lambda b: (0, jnp.minimum(b + GRID, NCB))),
        ],
        out_specs=pl.BlockSpec((RB, 128), lambda b: (b, 0)),
        out_shape=jax.ShapeDtypeStruct((HALF, 128), jnp.float32),
        compiler_params=pltpu.CompilerParams(fuse_transposed_lhs_in_matmul=True),
    )(wt_T, wt_T)


_mesh = plsc.VectorSubcoreMesh(core_axis_name="c", subcore_axis_name="s")


@functools.partial(
    pl.kernel,
    mesh=_mesh,
    out_type=jax.ShapeDtypeStruct((B, D), jnp.float32),
    scratch_types=[
        pltpu.VMEM((BPW,), jnp.int32),
        pltpu.VMEM((BPW // L, L), jnp.int32),
        pltpu.VMEM((BPW, 128), jnp.float32),
        pltpu.VMEM((BPW, D), jnp.float32),
        pltpu.VMEM((BPW, D), jnp.float32),
        pltpu.SemaphoreType.DMA,
        pltpu.SemaphoreType.DMA,
    ],
)
def _sc_gather(pair_hbm, half_hbm, tab2_hbm, pos_hbm, out_hbm,
               pair_v, half_v, rows_v, pos_v, out_v, gsem, psem):
    wid = lax.axis_index("s") * NC + lax.axis_index("c")
    base = wid * BPW
    pltpu.sync_copy(pair_hbm.at[pl.ds(base, BPW)], pair_v)
    pltpu.sync_copy(half_hbm.at[pl.ds(wid * (BPW // L), BPW // L)], half_v)
    pos_cp = pltpu.async_copy(pos_hbm.at[pl.ds(base, BPW)], pos_v, psem)
    for j in range(BPW // 128):
        sl = pl.ds(j * 128, 128)
        pltpu.async_copy(tab2_hbm.at[pair_v.at[sl]], rows_v.at[sl], gsem)
    pos_cp.wait()
    for j in range(BPW // 128):
        sl = pl.ds(j * 128, 128)
        pltpu.make_async_copy(tab2_hbm.at[pair_v.at[sl]], rows_v.at[sl],
                              gsem).wait()

    def gbody(g, _):
        hv = half_v[g]
        for j in range(L):
            r = g * L + j
            h = hv[j]

            @pl.when(h == 0)
            def _lo():
                for q in range(D // L):
                    sl = pl.ds(q * L, L)
                    out_v[r, sl] = rows_v[r, sl] + pos_v[r, sl]

            @pl.when(h != 0)
            def _hi():
                for q in range(D // L):
                    sl = pl.ds(q * L, L)
                    out_v[r, sl] = rows_v[r, pl.ds(D + q * L, L)] + pos_v[r, sl]

        return 0

    lax.fori_loop(0, BPW // L, gbody, 0)
    pltpu.sync_copy(out_v, out_hbm.at[pl.ds(base, BPW)])


def kernel(x, word_table, pos_table):
    xi = x.astype(jnp.int32)
    tab2 = _transpose_pack(word_table.T)
    return _sc_gather(xi & (HALF - 1), (xi >> 19).reshape(B // L, L), tab2,
                      pos_table[:B])


# speedup vs baseline: 1.1290x; 1.1290x over previous
import functools

import jax
import jax.numpy as jnp
from jax import lax
from jax.experimental import pallas as pl
from jax.experimental.pallas import tpu as pltpu
from jax.experimental.pallas import tpu_sc as plsc

V = 1000000
D = 64
B = 8192
NC, NS, L = 2, 16, 16
NW = NC * NS
BPW = B // NW
HALF = 1 << 19
RB = 8192
GRID = HALF // RB
NCB = (V + RB - 1) // RB - 1


def _transpose_pack(wt_T):
    def body(lo_ref, hi_ref, out_ref):
        eye = (jax.lax.broadcasted_iota(jnp.int32, (D, D), 0)
               == jax.lax.broadcasted_iota(jnp.int32, (D, D), 1)
               ).astype(jnp.float32)
        dn = (((0,), (0,)), ((), ()))
        lo_t = jax.lax.dot_general(lo_ref[...], eye, dn,
                                   preferred_element_type=jnp.float32)
        hi_t = jax.lax.dot_general(hi_ref[...], eye, dn,
                                   preferred_element_type=jnp.float32)
        out_ref[...] = jnp.concatenate([lo_t, hi_t], axis=1)

    return pl.pallas_call(
        body,
        grid=(GRID,),
        in_specs=[
            pl.BlockSpec((D, RB), lambda b: (0, b)),
            pl.BlockSpec((D, RB), lambda b: (0, jnp.minimum(b + GRID, NCB))),
        ],
        out_specs=pl.BlockSpec((RB, 128), lambda b: (b, 0)),
        out_shape=jax.ShapeDtypeStruct((HALF, 128), jnp.float32),
        compiler_params=pltpu.CompilerParams(fuse_transposed_lhs_in_matmul=True),
    )(wt_T, wt_T)


_mesh = plsc.VectorSubcoreMesh(core_axis_name="c", subcore_axis_name="s")


@functools.partial(
    pl.kernel,
    mesh=_mesh,
    out_type=jax.ShapeDtypeStruct((B, D), jnp.float32),
    scratch_types=[
        pltpu.VMEM((BPW,), jnp.int32),
        pltpu.VMEM((BPW // L, L), jnp.int32),
        pltpu.VMEM((BPW, 128), jnp.float32),
        pltpu.VMEM((BPW, D), jnp.float32),
        pltpu.VMEM((BPW, D), jnp.float32),
        pltpu.SemaphoreType.DMA,
        pltpu.SemaphoreType.DMA,
    ],
)
def _sc_gather(pair_hbm, half_hbm, tab2_hbm, pos_hbm, out_hbm,
               pair_v, half_v, rows_v, pos_v, out_v, gsem, psem):
    wid = lax.axis_index("s") * NC + lax.axis_index("c")
    base = wid * BPW
    pltpu.sync_copy(pair_hbm.at[pl.ds(base, BPW)], pair_v)
    pltpu.sync_copy(half_hbm.at[pl.ds(wid * (BPW // L), BPW // L)], half_v)
    pos_cp = pltpu.async_copy(pos_hbm.at[pl.ds(base, BPW)], pos_v, psem)
    for j in range(BPW // 128):
        sl = pl.ds(j * 128, 128)
        pltpu.async_copy(tab2_hbm.at[pair_v.at[sl]], rows_v.at[sl], gsem)
    pos_cp.wait()
    for j in range(BPW // 128):
        sl = pl.ds(j * 128, 128)
        pltpu.make_async_copy(tab2_hbm.at[pair_v.at[sl]], rows_v.at[sl],
                              gsem).wait()

    def gbody(g, _):
        hv = half_v[g]
        for j in range(L):
            r = g * L + j
            h = hv[j]

            @pl.when(h == 0)
            def _lo():
                for q in range(D // L):
                    sl = pl.ds(q * L, L)
                    out_v[r, sl] = rows_v[r, sl] + pos_v[r, sl]

            @pl.when(h != 0)
            def _hi():
                for q in range(D // L):
                    sl = pl.ds(q * L, L)
                    out_v[r, sl] = rows_v[r, pl.ds(D + q * L, L)] + pos_v[r, sl]

        return 0

    lax.fori_loop(0, BPW // L, gbody, 0)
    pltpu.sync_copy(out_v, out_hbm.at[pl.ds(base, BPW)])


def kernel(x, word_table, pos_table):
    xi = x.astype(jnp.int32)
    tab2 = _transpose_pack(word_table.T)
    return _sc_gather(xi & (HALF - 1), (xi >> 19).reshape(B // L, L), tab2,
                      pos_table[:B])


# speedup vs baseline: 1.1879x; 1.0521x over previous
import functools

import jax
import jax.numpy as jnp
from jax import lax
from jax.experimental import pallas as pl
from jax.experimental.pallas import tpu as pltpu
from jax.experimental.pallas import tpu_sc as plsc

V = 1000000
D = 64
B = 8192
NC, NS, L = 2, 16, 16
NW = NC * NS
BPW = B // NW
HALF = 1 << 19
RB = 16384
GRID = HALF // RB
NCB = (V + RB - 1) // RB - 1


def _transpose_pack(wt_T):
    def body(lo_ref, hi_ref, out_ref):
        eye = (jax.lax.broadcasted_iota(jnp.int32, (D, D), 0)
               == jax.lax.broadcasted_iota(jnp.int32, (D, D), 1)
               ).astype(jnp.float32)
        dn = (((0,), (0,)), ((), ()))
        out_ref[:, 0:D] = jax.lax.dot_general(
            lo_ref[...], eye, dn, preferred_element_type=jnp.float32)
        out_ref[:, D:128] = jax.lax.dot_general(
            hi_ref[...], eye, dn, preferred_element_type=jnp.float32)

    return pl.pallas_call(
        body,
        grid=(GRID,),
        in_specs=[
            pl.BlockSpec((D, RB), lambda b: (0, b)),
            pl.BlockSpec((D, RB), lambda b: (0, jnp.minimum(b + GRID, NCB))),
        ],
        out_specs=pl.BlockSpec((RB, 128), lambda b: (b, 0)),
        out_shape=jax.ShapeDtypeStruct((HALF, 128), jnp.float32),
        compiler_params=pltpu.CompilerParams(fuse_transposed_lhs_in_matmul=True),
    )(wt_T, wt_T)


_mesh = plsc.VectorSubcoreMesh(core_axis_name="c", subcore_axis_name="s")


@functools.partial(
    pl.kernel,
    mesh=_mesh,
    out_type=jax.ShapeDtypeStruct((B, D), jnp.float32),
    scratch_types=[
        pltpu.VMEM((BPW,), jnp.int32),
        pltpu.VMEM((BPW // L, L), jnp.int32),
        pltpu.VMEM((BPW, 128), jnp.float32),
        pltpu.VMEM((BPW, D), jnp.float32),
        pltpu.VMEM((BPW, D), jnp.float32),
        pltpu.SemaphoreType.DMA,
        pltpu.SemaphoreType.DMA,
    ],
)
def _sc_gather(pair_hbm, half_hbm, tab2_hbm, pos_hbm, out_hbm,
               pair_v, half_v, rows_v, pos_v, out_v, gsem, psem):
    wid = lax.axis_index("s") * NC + lax.axis_index("c")
    base = wid * BPW
    pltpu.sync_copy(pair_hbm.at[pl.ds(base, BPW)], pair_v)
    pltpu.sync_copy(half_hbm.at[pl.ds(wid * (BPW // L), BPW // L)], half_v)
    pos_cp = pltpu.async_copy(pos_hbm.at[pl.ds(base, BPW)], pos_v, psem)
    for j in range(BPW // 128):
        sl = pl.ds(j * 128, 128)
        pltpu.async_copy(tab2_hbm.at[pair_v.at[sl]], rows_v.at[sl], gsem)
    pos_cp.wait()
    for j in range(BPW // 128):
        sl = pl.ds(j * 128, 128)
        pltpu.make_async_copy(tab2_hbm.at[pair_v.at[sl]], rows_v.at[sl],
                              gsem).wait()

    def gbody(g, _):
        hv = half_v[g]
        for j in range(L):
            r = g * L + j
            h = hv[j]

            @pl.when(h == 0)
            def _lo():
                for q in range(D // L):
                    sl = pl.ds(q * L, L)
                    out_v[r, sl] = rows_v[r, sl] + pos_v[r, sl]

            @pl.when(h != 0)
            def _hi():
                for q in range(D // L):
                    sl = pl.ds(q * L, L)
                    out_v[r, sl] = rows_v[r, pl.ds(D + q * L, L)] + pos_v[r, sl]

        return 0

    lax.fori_loop(0, BPW // L, gbody, 0)
    pltpu.sync_copy(out_v, out_hbm.at[pl.ds(base, BPW)])


def kernel(x, word_table, pos_table):
    xi = x.astype(jnp.int32)
    tab2 = _transpose_pack(word_table.T)
    return _sc_gather(xi & (HALF - 1), (xi >> 19).reshape(B // L, L), tab2,
                      pos_table[:B])


# speedup vs baseline: 1.3699x; 1.1532x over previous
import functools

import jax
import jax.numpy as jnp
from jax import lax
from jax.experimental import pallas as pl
from jax.experimental.pallas import tpu as pltpu
from jax.experimental.pallas import tpu_sc as plsc

V = 1000000
D = 64
B = 8192
NC, NS, L = 2, 16, 16
NW = NC * NS
BPW = B // NW
E = 1 << 18
RB = 8192
GRID = E // RB
KOFF = E // RB
NCB = (V + RB - 1) // RB - 1


def _transpose_pack(wt_T):
    def body(q0_ref, q1_ref, q2_ref, q3_ref, out_ref):
        eye = (jax.lax.broadcasted_iota(jnp.int32, (D, D), 0)
               == jax.lax.broadcasted_iota(jnp.int32, (D, D), 1)
               ).astype(jnp.float32)
        dn = (((0,), (0,)), ((), ()))

        def tbits(ref):
            t = jax.lax.dot_general(ref[...], eye, dn,
                                    preferred_element_type=jnp.float32)
            b = t.astype(jnp.bfloat16)
            return jax.lax.bitcast_convert_type(b, jnp.uint16).astype(
                jnp.uint32)

        w01 = tbits(q0_ref) | (tbits(q1_ref) << 16)
        w23 = tbits(q2_ref) | (tbits(q3_ref) << 16)
        out_ref[:, 0:D] = jax.lax.bitcast_convert_type(w01, jnp.float32)
        out_ref[:, D:128] = jax.lax.bitcast_convert_type(w23, jnp.float32)

    return pl.pallas_call(
        body,
        grid=(GRID,),
        in_specs=[
            pl.BlockSpec((D, RB),
                         lambda b, k=k: (0, jnp.minimum(b + k * KOFF, NCB)))
            for k in range(4)
        ],
        out_specs=pl.BlockSpec((RB, 128), lambda b: (b, 0)),
        out_shape=jax.ShapeDtypeStruct((E, 128), jnp.float32),
        compiler_params=pltpu.CompilerParams(fuse_transposed_lhs_in_matmul=True),
    )(wt_T, wt_T, wt_T, wt_T)


_mesh = plsc.VectorSubcoreMesh(core_axis_name="c", subcore_axis_name="s")


@functools.partial(
    pl.kernel,
    mesh=_mesh,
    out_type=jax.ShapeDtypeStruct((B, D), jnp.float32),
    scratch_types=[
        pltpu.VMEM((BPW,), jnp.int32),
        pltpu.VMEM((BPW // L, L), jnp.int32),
        pltpu.VMEM((BPW, 128), jnp.float32),
        pltpu.VMEM((BPW, D), jnp.float32),
        pltpu.VMEM((BPW, D), jnp.float32),
        pltpu.SemaphoreType.DMA,
        pltpu.SemaphoreType.DMA,
    ],
    compiler_params=pltpu.CompilerParams(needs_layout_passes=False),
)
def _sc_gather(pair_hbm, quad_hbm, tab2_hbm, pos_hbm, out_hbm,
               pair_v, quad_v, rows_v, pos_v, out_v, gsem, psem):
    wid = lax.axis_index("s") * NC + lax.axis_index("c")
    base = wid * BPW
    pltpu.sync_copy(pair_hbm.at[pl.ds(base, BPW)], pair_v)
    pltpu.sync_copy(quad_hbm.at[pl.ds(wid * (BPW // L), BPW // L)], quad_v)
    pos_cp = pltpu.async_copy(pos_hbm.at[pl.ds(base, BPW)], pos_v, psem)
    for j in range(BPW // 128):
        sl = pl.ds(j * 128, 128)
        pltpu.async_copy(tab2_hbm.at[pair_v.at[sl]], rows_v.at[sl], gsem)
    pos_cp.wait()
    for j in range(BPW // 128):
        sl = pl.ds(j * 128, 128)
        pltpu.make_async_copy(tab2_hbm.at[pair_v.at[sl]], rows_v.at[sl],
                              gsem).wait()

    def emit(r, lane_base, take_high):
        for q in range(D // L):
            sl = pl.ds(q * L, L)
            w = rows_v[r, pl.ds(lane_base + q * L, L)]
            ab = plsc.bitcast(w, jnp.bfloat16)
            lo, hi = plsc.unpack(ab, format=plsc.PackFormat.INTERLEAVED)
            val = hi if take_high else lo
            out_v[r, sl] = val + pos_v[r, sl]

    def gbody(g, _):
        qv = quad_v[g]
        for j in range(L):
            r = g * L + j
            e = qv[j]
            for eq in range(4):
                @pl.when(e == eq)
                def _(eq=eq):
                    emit(r, (eq // 2) * D, bool(eq % 2))

        return 0

    lax.fori_loop(0, BPW // L, gbody, 0)
    pltpu.sync_copy(out_v, out_hbm.at[pl.ds(base, BPW)])


def kernel(x, word_table, pos_table):
    xi = x.astype(jnp.int32)
    tab2 = _transpose_pack(word_table.T)
    return _sc_gather(xi & (E - 1), (xi >> 18).reshape(B // L, L), tab2,
                      pos_table[:B])


# speedup vs baseline: 1.7710x; 1.2928x over previous
import functools

import jax
import jax.numpy as jnp
from jax import lax
from jax.experimental import pallas as pl
from jax.experimental.pallas import tpu as pltpu
from jax.experimental.pallas import tpu_sc as plsc

V = 1000000
D = 64
B = 8192
NC, NS, L = 2, 16, 16
NW = NC * NS
BPW = B // NW
E = 1 << 18
RB = 16384
GRID = E // RB
KOFF = E // RB
NCB = (V + RB - 1) // RB - 1


def _transpose_pack(wt_T):
    def body(q0_ref, q1_ref, q2_ref, q3_ref, out_ref):
        eye = (jax.lax.broadcasted_iota(jnp.int32, (D, D), 0)
               == jax.lax.broadcasted_iota(jnp.int32, (D, D), 1)
               ).astype(jnp.bfloat16)
        dn = (((0,), (0,)), ((), ()))

        def tbits(ref):
            t = jax.lax.dot_general(ref[...].astype(jnp.bfloat16), eye, dn,
                                    preferred_element_type=jnp.float32)
            return jax.lax.bitcast_convert_type(t.astype(jnp.bfloat16),
                                                jnp.uint16).astype(jnp.uint32)

        w01 = tbits(q0_ref) | (tbits(q1_ref) << 16)
        w23 = tbits(q2_ref) | (tbits(q3_ref) << 16)
        out_ref[:, 0:D] = jax.lax.bitcast_convert_type(w01, jnp.float32)
        out_ref[:, D:128] = jax.lax.bitcast_convert_type(w23, jnp.float32)

    return pl.pallas_call(
        body,
        grid=(GRID,),
        in_specs=[
            pl.BlockSpec((D, RB),
                         lambda b, k=k: (0, jnp.minimum(b + k * KOFF, NCB)))
            for k in range(4)
        ],
        out_specs=pl.BlockSpec((RB, 128), lambda b: (b, 0)),
        out_shape=jax.ShapeDtypeStruct((E, 128), jnp.float32),
        compiler_params=pltpu.CompilerParams(fuse_transposed_lhs_in_matmul=True),
    )(wt_T, wt_T, wt_T, wt_T)


_mesh = plsc.VectorSubcoreMesh(core_axis_name="c", subcore_axis_name="s")


@functools.partial(
    pl.kernel,
    mesh=_mesh,
    out_type=jax.ShapeDtypeStruct((B, D), jnp.float32),
    scratch_types=[
        pltpu.VMEM((BPW,), jnp.int32),
        pltpu.VMEM((BPW // L, L), jnp.int32),
        pltpu.VMEM((BPW, 128), jnp.float32),
        pltpu.VMEM((BPW, D), jnp.float32),
        pltpu.VMEM((BPW, D), jnp.float32),
        pltpu.SemaphoreType.DMA,
        pltpu.SemaphoreType.DMA,
    ],
    compiler_params=pltpu.CompilerParams(needs_layout_passes=False),
)
def _sc_gather(pair_hbm, quad_hbm, tab2_hbm, pos_hbm, out_hbm,
               pair_v, quad_v, rows_v, pos_v, out_v, gsem, psem):
    wid = lax.axis_index("s") * NC + lax.axis_index("c")
    base = wid * BPW
    pltpu.sync_copy(pair_hbm.at[pl.ds(base, BPW)], pair_v)
    pltpu.sync_copy(quad_hbm.at[pl.ds(wid * (BPW // L), BPW // L)], quad_v)
    pos_cp = pltpu.async_copy(pos_hbm.at[pl.ds(base, BPW)], pos_v, psem)
    for j in range(BPW // 128):
        sl = pl.ds(j * 128, 128)
        pltpu.async_copy(tab2_hbm.at[pair_v.at[sl]], rows_v.at[sl], gsem)
    pos_cp.wait()
    for j in range(BPW // 128):
        sl = pl.ds(j * 128, 128)
        pltpu.make_async_copy(tab2_hbm.at[pair_v.at[sl]], rows_v.at[sl],
                              gsem).wait()

    def emit(r, lane_base, take_high):
        for q in range(D // L):
            sl = pl.ds(q * L, L)
            w = rows_v[r, pl.ds(lane_base + q * L, L)]
            ab = plsc.bitcast(w, jnp.bfloat16)
            lo, hi = plsc.unpack(ab, format=plsc.PackFormat.INTERLEAVED)
            val = hi if take_high else lo
            out_v[r, sl] = val + pos_v[r, sl]

    def gbody(g, _):
        qv = quad_v[g]
        for j in range(L):
            r = g * L + j
            e = qv[j]
            for eq in range(4):
                @pl.when(e == eq)
                def _(eq=eq):
                    emit(r, (eq // 2) * D, bool(eq % 2))

        return 0

    lax.fori_loop(0, BPW // L, gbody, 0)
    pltpu.sync_copy(out_v, out_hbm.at[pl.ds(base, BPW)])


def kernel(x, word_table, pos_table):
    xi = x.astype(jnp.int32)
    tab2 = _transpose_pack(word_table.T)
    return _sc_gather(xi & (E - 1), (xi >> 18).reshape(B // L, L), tab2,
                      pos_table[:B])
